# baseline (speedup 1.0000x reference)
import functools

import jax
import jax.numpy as jnp
from jax import lax
from jax.experimental import pallas as pl
from jax.experimental.pallas import tpu as pltpu
from jax.experimental.pallas import tpu_sc as plsc

NEG_SLOPE = 0.2
LANES = 16
NR = 12512
CF = 128
CS = 1600

_MESH = dict(core_axis_name="c", subcore_axis_name="s")
_CP = pltpu.CompilerParams(needs_layout_passes=False)


def _elu(v):
    return jnp.where(v > 0, v, jnp.exp(jnp.minimum(v, 0.0)) - 1.0)


def _tc1_body(x_ref, w_ref, al_ref, ar_ref, t_ref, el_ref, er_ref):
    feat = jnp.dot(x_ref[...], w_ref[...], preferred_element_type=jnp.float32)
    bn = feat.shape[0]
    t_ref[...] = jnp.concatenate(
        [feat, jnp.ones((bn, 2), jnp.float32), jnp.zeros((bn, 62), jnp.float32)],
        axis=1)
    el_ref[...] = jnp.dot(feat, al_ref[...], preferred_element_type=jnp.float32)
    er_ref[...] = jnp.dot(feat, ar_ref[...], preferred_element_type=jnp.float32)


def _tc1(x, W1, almat, armat, bn=1000):
    n, din = x.shape
    h = almat.shape[1]
    full = lambda a: pl.BlockSpec(a.shape, lambda i: (0,) * a.ndim)
    return pl.pallas_call(
        _tc1_body,
        grid=(n // bn,),
        in_specs=[pl.BlockSpec((bn, din), lambda i: (i, 0)),
                  full(W1), full(almat), full(armat)],
        out_specs=[pl.BlockSpec((bn, 128), lambda i: (i, 0)),
                   pl.BlockSpec((bn, h), lambda i: (i, 0)),
                   pl.BlockSpec((bn, h), lambda i: (i, 0))],
        out_shape=[jax.ShapeDtypeStruct((n, 128), jnp.float32),
                   jax.ShapeDtypeStruct((n, h), jnp.float32),
                   jax.ShapeDtypeStruct((n, h), jnp.float32)],
    )(x, W1, almat, armat)


def _tc2_body(acc_ref, b1_ref, w2_ref, al_ref, ar_ref, t_ref, el_ref, er_ref):
    a0 = acc_ref[:, 0:32]
    a1 = acc_ref[:, 32:64]
    d0 = acc_ref[:, 64:65]
    d1 = acc_ref[:, 65:66]
    h0 = _elu(jnp.where(d0 > 0, a0 / d0, 0.0) + b1_ref[:, :32])
    h1 = _elu(jnp.where(d1 > 0, a1 / d1, 0.0) + b1_ref[:, 32:])
    h = jnp.concatenate([h0, h1], axis=1)
    feat = jnp.dot(h, w2_ref[...], preferred_element_type=jnp.float32)
    t_ref[...] = feat
    el_ref[...] = jnp.dot(feat, al_ref[...], preferred_element_type=jnp.float32)
    er_ref[...] = jnp.dot(feat, ar_ref[...], preferred_element_type=jnp.float32)


def _tc2(acc, b1r, W2, almat, armat, bn=1000):
    n = acc.shape[0]
    full = lambda a: pl.BlockSpec(a.shape, lambda i: (0,) * a.ndim)
    return pl.pallas_call(
        _tc2_body,
        grid=(n // bn,),
        in_specs=[pl.BlockSpec((bn, 128), lambda i: (i, 0)),
                  full(b1r), full(W2), full(almat), full(armat)],
        out_specs=[pl.BlockSpec((bn, 128), lambda i: (i, 0)),
                   pl.BlockSpec((bn, 1), lambda i: (i, 0)),
                   pl.BlockSpec((bn, 1), lambda i: (i, 0))],
        out_shape=[jax.ShapeDtypeStruct((n, 128), jnp.float32),
                   jax.ShapeDtypeStruct((n, 1), jnp.float32),
                   jax.ShapeDtypeStruct((n, 1), jnp.float32)],
    )(acc, b1r, W2, almat, armat)


def _tc3_body(acc_ref, dnm_ref, b2_ref, out_ref):
    d = (dnm_ref[:, 0] + dnm_ref[:, 1])[:, None]
    out_ref[...] = _elu(jnp.where(d > 0, acc_ref[...] / d, 0.0) + b2_ref[...])


def _tc3(acc, dnm, b2r, bn=1000):
    n = acc.shape[0]
    full = lambda a: pl.BlockSpec(a.shape, lambda i: (0,) * a.ndim)
    return pl.pallas_call(
        _tc3_body,
        grid=(n // bn,),
        in_specs=[pl.BlockSpec((bn, 128), lambda i: (i, 0)),
                  pl.BlockSpec((bn, 2), lambda i: (i, 0)), full(b2r)],
        out_specs=pl.BlockSpec((bn, 128), lambda i: (i, 0)),
        out_shape=jax.ShapeDtypeStruct((n, 128), jnp.float32),
    )(acc, dnm, b2r)


def _tile_rows(n):
    even = ((n + 15) // 16 + 7) // 8 * 8
    return even, n - 15 * even


def _zero_vec(ref, c):
    def body(i, _):
        ref[pl.ds(i * LANES, LANES)] = jnp.zeros((LANES,), jnp.float32)
        return 0
    lax.fori_loop(0, c // LANES, body, 0)


def _zero_ivec(ref, c):
    def body(i, _):
        ref[pl.ds(i * LANES, LANES)] = jnp.zeros((LANES,), jnp.int32)
        return 0
    lax.fori_loop(0, c // LANES, body, 0)


def _slab_copy(src_at, dst_at, even, tail, s, chunk, via_at=None):
    def emit(total, base):
        off = 0
        while off < total:
            sz = min(chunk, total - off)
            if via_at is None:
                pltpu.sync_copy(src_at(base + off, sz), dst_at(base + off, sz))
            else:
                pltpu.sync_copy(src_at(base + off, sz), via_at(sz))
                pltpu.sync_copy(via_at(sz), dst_at(base + off, sz))
            off += sz

    @pl.when(s < 15)
    def _():
        emit(even, s * even)

    @pl.when(s == 15)
    def _():
        emit(tail, 15 * even)


def _make_attn(K, N, E, with_dnm):
    C = CS
    even, tail = _tile_rows(N)
    nch = E // C
    assert E % C == 0 and C % LANES == 0

    def body(*refs):
        srcs, dsts = refs[0], refs[1]
        els = refs[2:2 + K]
        ers = refs[2 + K:2 + 2 * K]
        cvec = refs[2 + 2 * K]
        ee_out = refs[3 + 2 * K]
        i = 5 + 2 * K if with_dnm else 4 + 2 * K
        dnm_out = refs[4 + 2 * K] if with_dnm else None
        src_v, dst_v = refs[i], refs[i + 1]
        elgs = refs[i + 2:i + 2 + K]
        ergs = refs[i + 2 + K:i + 2 + 2 * K]
        eevs = refs[i + 2 + 2 * K:i + 2 + 3 * K]
        cv, sem = refs[i + 2 + 3 * K], refs[i + 3 + 3 * K]
        dnm_sh = refs[i + 4 + 3 * K:i + 4 + 4 * K] if with_dnm else ()

        c = lax.axis_index("c")
        s = lax.axis_index("s")
        wid = c * 16 + s

        if with_dnm:
            _zero_vec(eevs[0], C)
            for k in range(K):
                _slab_copy(lambda o, z: eevs[0].at[pl.ds(0, z)],
                           lambda o, z, k=k: dnm_sh[k].at[pl.ds(o, z)],
                           even, tail, s, chunk=C)
            plsc.subcore_barrier()

        cvvs = []
        for k in range(K):
            pltpu.sync_copy(cvec.at[pl.ds(k * LANES, LANES)], cv)
            cvvs.append(cv[...])

        def chunk(j, _):
            base = (wid + j * 32) * C
            d1 = pltpu.async_copy(srcs.at[pl.ds(base, C)], src_v, sem)
            d2 = pltpu.async_copy(dsts.at[pl.ds(base, C)], dst_v, sem)
            d1.wait()
            d2.wait()
            dg = [pltpu.async_copy(els[k].at[src_v], elgs[k], sem)
                  for k in range(K)]
            dg += [pltpu.async_copy(ers[k].at[dst_v], ergs[k], sem)
                   for k in range(K)]
            for d in dg:
                d.wait()

            for k in range(K):
                cvv = cvvs[k]

                @plsc.parallel_loop(0, C // LANES, unroll=2)
                def _vr(i2, k=k, cvv=cvv):
                    z = (elgs[k][pl.ds(i2 * LANES, LANES)]
                         + ergs[k][pl.ds(i2 * LANES, LANES)])
                    z = jnp.where(z >= 0, z, NEG_SLOPE * z)
                    eevs[k][pl.ds(i2 * LANES, LANES)] = jnp.exp(z - cvv)

                pltpu.sync_copy(eevs[k], ee_out.at[pl.ds(k * E + base, C)])
                if with_dnm:
                    pltpu.sync_copy(eevs[k], dnm_sh[k].at[dst_v], add=True)
            return 0
        lax.fori_loop(0, (nch - wid + 31) // 32, chunk, 0)

        if with_dnm:
            plsc.subcore_barrier()
            for k in range(K):
                _slab_copy(lambda o, z, k=k: dnm_sh[k].at[pl.ds(o, z)],
                           lambda o, z, k=k: dnm_out.at[pl.ds((c * K + k) * N + o, z)],
                           even, tail, s, chunk=C,
                           via_at=lambda z: eevs[0].at[pl.ds(0, z)])

    out_type = [jax.ShapeDtypeStruct((K * E,), jnp.float32)]
    if with_dnm:
        out_type.append(jax.ShapeDtypeStruct((2 * K * N,), jnp.float32))
    scratch = ([pltpu.VMEM((C,), jnp.int32),
                pltpu.VMEM((C,), jnp.int32)]
               + [pltpu.VMEM((C,), jnp.float32)] * (3 * K)
               + [pltpu.VMEM((LANES,), jnp.float32),
                  pltpu.SemaphoreType.DMA])
    if with_dnm:
        scratch += [pltpu.VMEM_SHARED((N,), jnp.float32)] * K
    return pl.kernel(body, out_type=out_type,
                     mesh=plsc.VectorSubcoreMesh(**_MESH),
                     scratch_types=scratch, compiler_params=_CP)


def _make_agg(KE, N, E, phase, segsets):
    C = CS
    even, tail = _tile_rows(NR)
    nch = E // C
    assert E % C == 0

    def body(*refs):
        srcs, dsts, tbl = refs[0], refs[1], refs[2]
        eerows = refs[3]
        acc_out = refs[4]
        src_vs = refs[5:7]
        dst_vs = refs[7:9]
        eevss = (refs[9:9 + KE], refs[9 + KE:9 + 2 * KE])
        r = 9 + 2 * KE
        st_src = refs[r]
        st_dst = refs[r + 1]
        st_es = refs[r + 2:r + 2 + KE]
        fire_src = refs[r + 2 + KE]
        fire_dst = refs[r + 3 + KE]
        rows_v = refs[r + 4 + KE]
        acc_sh = refs[r + 5 + KE]
        sem = refs[r + 6 + KE]

        c = lax.axis_index("c")
        s = lax.axis_index("s")
        lo = (2 * phase + c) * NR

        def zrows(i, _):
            for half in range(8):
                rows_v[i, pl.ds(half * LANES, LANES)] = jnp.zeros((LANES,), jnp.float32)
            return 0
        lax.fori_loop(0, CF, zrows, 0)
        _slab_copy(lambda o, z: rows_v.at[pl.ds(0, z)],
                   lambda o, z: acc_sh.at[pl.ds(o, z)],
                   even, tail, s, chunk=CF)
        _zero_ivec(st_src, CF + LANES)
        _zero_ivec(st_dst, CF + LANES)
        for k in range(KE):
            _zero_vec(st_es[k], CF + LANES)
        plsc.subcore_barrier()

        iota = lax.iota(jnp.int32, LANES)

        def fire():
            def mv(i, _):
                fire_src[pl.ds(i * LANES, LANES)] = st_src[pl.ds(i * LANES, LANES)]
                fire_dst[pl.ds(i * LANES, LANES)] = st_dst[pl.ds(i * LANES, LANES)]
                return 0
            lax.fori_loop(0, CF // LANES, mv, 0)
            pltpu.sync_copy(tbl.at[fire_src], rows_v)

            @plsc.parallel_loop(0, CF // LANES, unroll=2)
            def _scale(i):
                ees = [st_es[k][pl.ds(i * LANES, LANES)] for k in range(KE)]
                for t in range(LANES):
                    tv = jnp.full((LANES,), t, jnp.int32)
                    sp = [e.at[tv].get(mode="promise_in_bounds") for e in ees]
                    for seg, k in segsets(sp):
                        r = rows_v[i * LANES + t, pl.ds(seg * LANES, LANES)]
                        rows_v[i * LANES + t, pl.ds(seg * LANES, LANES)] = r * k
            pltpu.sync_copy(rows_v, acc_sh.at[fire_dst], add=True)

        ntrips = (nch - s + 15) // 16

        def issue(j, b):
            base = (s + j * 16) * C
            return ([pltpu.async_copy(srcs.at[pl.ds(base, C)], src_vs[b], sem),
                     pltpu.async_copy(dsts.at[pl.ds(base, C)], dst_vs[b], sem)]
                    + [pltpu.async_copy(eerows.at[pl.ds(k * E + base, C)],
                                        eevss[b][k], sem) for k in range(KE)])

        def process(b, pos):
            src_v, dst_v, eevs = src_vs[b], dst_vs[b], eevss[b]

            def vr(i, pos):
                sv = src_v[pl.ds(i * LANES, LANES)]
                dv = dst_v[pl.ds(i * LANES, LANES)]
                dl = dv - lo
                m = (dl >= 0) & (dl < NR)
                plsc.store_compressed(st_src.at[pl.ds(pos, LANES)], sv, mask=m)
                plsc.store_compressed(st_dst.at[pl.ds(pos, LANES)], dl, mask=m)
                for k in range(KE):
                    ev = eevs[k][pl.ds(i * LANES, LANES)]
                    plsc.store_compressed(st_es[k].at[pl.ds(pos, LANES)], ev, mask=m)
                pos2 = pos + jnp.sum(m.astype(jnp.int32))

                @pl.when(pos2 >= CF)
                def _():
                    fire()
                    v = st_src[pl.ds(CF, LANES)]
                    st_src[pl.ds(0, LANES)] = v
                    v = st_dst[pl.ds(CF, LANES)]
                    st_dst[pl.ds(0, LANES)] = v
                    for k in range(KE):
                        v = st_es[k][pl.ds(CF, LANES)]
                        st_es[k][pl.ds(0, LANES)] = v
                return jnp.where(pos2 >= CF, pos2 - CF, pos2)
            return lax.fori_loop(0, C // LANES, vr, pos)

        def pair(p, pos):
            j0 = 2 * p
            d0 = issue(j0, 0)
            has2 = j0 + 1 < ntrips

            @pl.when(has2)
            def _():
                issue(j0 + 1, 1)
            for d in d0:
                d.wait()
            pos = process(0, pos)

            def second(pos):
                pltpu.make_async_copy(srcs.at[pl.ds(0, C)], src_vs[1], sem).wait()
                pltpu.make_async_copy(dsts.at[pl.ds(0, C)], dst_vs[1], sem).wait()
                for k in range(KE):
                    pltpu.make_async_copy(eerows.at[pl.ds(0, C)],
                                          eevss[1][k], sem).wait()
                return process(1, pos)
            return lax.cond(has2, second, lambda pos: pos, pos)

        pos = lax.fori_loop(0, (ntrips + 1) // 2, pair, 0)

        def ztail(i, p):
            gi = i * LANES + iota
            keep = gi < p
            for k in range(KE):
                v = st_es[k][pl.ds(i * LANES, LANES)]
                st_es[k][pl.ds(i * LANES, LANES)] = jnp.where(keep, v, 0.0)
            return p
        lax.fori_loop(0, CF // LANES, ztail, pos)
        fire()

        plsc.subcore_barrier()
        _slab_copy(lambda o, z: acc_sh.at[pl.ds(o, z)],
                   lambda o, z: acc_out.at[c, pl.ds(o, z)],
                   even, tail, s, chunk=CF,
                   via_at=lambda z: rows_v.at[pl.ds(0, z)])

    scratch = ([pltpu.VMEM((C,), jnp.int32)] * 2
               + [pltpu.VMEM((C,), jnp.int32)] * 2
               + [pltpu.VMEM((C,), jnp.float32)] * (2 * KE)
               + [pltpu.VMEM((CF + LANES,), jnp.int32),
                  pltpu.VMEM((CF + LANES,), jnp.int32)]
               + [pltpu.VMEM((CF + LANES,), jnp.float32)] * KE
               + [pltpu.VMEM((CF,), jnp.int32),
                  pltpu.VMEM((CF,), jnp.int32),
                  pltpu.VMEM((CF, 128), jnp.float32),
                  pltpu.VMEM_SHARED((NR, 128), jnp.float32),
                  pltpu.SemaphoreType.DMA])
    return pl.kernel(body,
                     out_type=jax.ShapeDtypeStruct((2, NR, 128), jnp.float32),
                     mesh=plsc.VectorSubcoreMesh(**_MESH),
                     scratch_types=scratch, compiler_params=_CP)


def _headmat(al, heads, dim):
    m = jnp.zeros((heads * dim, heads), jnp.float32)
    return m.at[jnp.arange(heads * dim), jnp.repeat(jnp.arange(heads), dim)
                ].set(al.reshape(-1))


def _bound(el, er):
    z = jnp.max(el, axis=0) + jnp.max(er, axis=0)
    return jnp.where(z >= 0, z, NEG_SLOPE * z)


def _l1_segsets(sp):
    lane = lax.iota(jnp.int32, LANES)
    mix = jnp.where(lane == 0, sp[0], jnp.where(lane == 1, sp[1], 0.0))
    return [(0, sp[0]), (1, sp[0]), (2, sp[1]), (3, sp[1]), (4, mix)]


def _l2_segsets(sp):
    return [(seg, sp[0]) for seg in range(8)]


def kernel(x, edge_index, W1, al1, ar1, b1, W2, al2, ar2, b2):
    n = x.shape[0]
    e = edge_index.shape[1]
    ei = edge_index.astype(jnp.int32)
    srcs, dsts = ei[0], ei[1]

    t1, el, er = _tc1(x, W1, _headmat(al1, 2, 32), _headmat(ar1, 2, 32))
    cvec1 = jnp.repeat(_bound(el, er), LANES)

    (ee1,) = _make_attn(2, n, e, with_dnm=False)(
        srcs, dsts, el[:, 0], el[:, 1], er[:, 0], er[:, 1], cvec1)

    ph0 = _make_agg(2, n, e, 0, _l1_segsets)(srcs, dsts, t1, ee1)
    ph1 = _make_agg(2, n, e, 1, _l1_segsets)(srcs, dsts, t1, ee1)
    acc1 = jnp.concatenate([ph0[0], ph0[1], ph1[0], ph1[1]], axis=0)[:n]

    t2, el2, er2 = _tc2(acc1, b1.reshape(1, -1), W2,
                        _headmat(al2, 1, 128), _headmat(ar2, 1, 128))
    cvec2 = jnp.repeat(_bound(el2, er2), LANES)

    ee2, dnm2 = _make_attn(1, n, e, with_dnm=True)(
        srcs, dsts, el2[:, 0], er2[:, 0], cvec2)

    qh0 = _make_agg(1, n, e, 0, _l2_segsets)(srcs, dsts, t2, ee2)
    qh1 = _make_agg(1, n, e, 1, _l2_segsets)(srcs, dsts, t2, ee2)
    acc2 = jnp.concatenate([qh0[0], qh0[1], qh1[0], qh1[1]], axis=0)[:n]

    dnm2t = dnm2.reshape(2, n).transpose(1, 0)
    return _tc3(acc2, dnm2t, b2.reshape(1, -1))


# speedup vs baseline: 1.0310x; 1.0310x over previous
import functools

import jax
import jax.numpy as jnp
from jax import lax
from jax.experimental import pallas as pl
from jax.experimental.pallas import tpu as pltpu
from jax.experimental.pallas import tpu_sc as plsc

NEG_SLOPE = 0.2
LANES = 16
NR = 12512
CF = 128
CS = 1600

_MESH = dict(core_axis_name="c", subcore_axis_name="s")
_CP = pltpu.CompilerParams(needs_layout_passes=False)


def _elu(v):
    return jnp.where(v > 0, v, jnp.exp(jnp.minimum(v, 0.0)) - 1.0)


def _tc1_body(x_ref, w_ref, al_ref, ar_ref, t_ref, el_ref, er_ref):
    feat = jnp.dot(x_ref[...], w_ref[...], preferred_element_type=jnp.float32)
    bn = feat.shape[0]
    t_ref[...] = jnp.concatenate(
        [feat, jnp.ones((bn, 2), jnp.float32), jnp.zeros((bn, 62), jnp.float32)],
        axis=1)
    el_ref[...] = jnp.dot(feat, al_ref[...], preferred_element_type=jnp.float32)
    er_ref[...] = jnp.dot(feat, ar_ref[...], preferred_element_type=jnp.float32)


def _tc1(x, W1, almat, armat, bn=1000):
    n, din = x.shape
    h = almat.shape[1]
    full = lambda a: pl.BlockSpec(a.shape, lambda i: (0,) * a.ndim)
    return pl.pallas_call(
        _tc1_body,
        grid=(n // bn,),
        in_specs=[pl.BlockSpec((bn, din), lambda i: (i, 0)),
                  full(W1), full(almat), full(armat)],
        out_specs=[pl.BlockSpec((bn, 128), lambda i: (i, 0)),
                   pl.BlockSpec((bn, h), lambda i: (i, 0)),
                   pl.BlockSpec((bn, h), lambda i: (i, 0))],
        out_shape=[jax.ShapeDtypeStruct((n, 128), jnp.float32),
                   jax.ShapeDtypeStruct((n, h), jnp.float32),
                   jax.ShapeDtypeStruct((n, h), jnp.float32)],
    )(x, W1, almat, armat)


def _tc2_body(acc_ref, b1_ref, w2_ref, al_ref, ar_ref, t_ref, el_ref, er_ref):
    a0 = acc_ref[:, 0:32]
    a1 = acc_ref[:, 32:64]
    d0 = acc_ref[:, 64:65]
    d1 = acc_ref[:, 65:66]
    h0 = _elu(jnp.where(d0 > 0, a0 / d0, 0.0) + b1_ref[:, :32])
    h1 = _elu(jnp.where(d1 > 0, a1 / d1, 0.0) + b1_ref[:, 32:])
    h = jnp.concatenate([h0, h1], axis=1)
    feat = jnp.dot(h, w2_ref[...], preferred_element_type=jnp.float32)
    t_ref[...] = feat
    el_ref[...] = jnp.dot(feat, al_ref[...], preferred_element_type=jnp.float32)
    er_ref[...] = jnp.dot(feat, ar_ref[...], preferred_element_type=jnp.float32)


def _tc2(acc, b1r, W2, almat, armat, bn=1000):
    n = acc.shape[0]
    full = lambda a: pl.BlockSpec(a.shape, lambda i: (0,) * a.ndim)
    return pl.pallas_call(
        _tc2_body,
        grid=(n // bn,),
        in_specs=[pl.BlockSpec((bn, 128), lambda i: (i, 0)),
                  full(b1r), full(W2), full(almat), full(armat)],
        out_specs=[pl.BlockSpec((bn, 128), lambda i: (i, 0)),
                   pl.BlockSpec((bn, 1), lambda i: (i, 0)),
                   pl.BlockSpec((bn, 1), lambda i: (i, 0))],
        out_shape=[jax.ShapeDtypeStruct((n, 128), jnp.float32),
                   jax.ShapeDtypeStruct((n, 1), jnp.float32),
                   jax.ShapeDtypeStruct((n, 1), jnp.float32)],
    )(acc, b1r, W2, almat, armat)


def _tc3_body(acc_ref, dnm_ref, b2_ref, out_ref):
    d = (dnm_ref[:, 0] + dnm_ref[:, 1])[:, None]
    out_ref[...] = _elu(jnp.where(d > 0, acc_ref[...] / d, 0.0) + b2_ref[...])


def _tc3(acc, dnm, b2r, bn=1000):
    n = acc.shape[0]
    full = lambda a: pl.BlockSpec(a.shape, lambda i: (0,) * a.ndim)
    return pl.pallas_call(
        _tc3_body,
        grid=(n // bn,),
        in_specs=[pl.BlockSpec((bn, 128), lambda i: (i, 0)),
                  pl.BlockSpec((bn, 2), lambda i: (i, 0)), full(b2r)],
        out_specs=pl.BlockSpec((bn, 128), lambda i: (i, 0)),
        out_shape=jax.ShapeDtypeStruct((n, 128), jnp.float32),
    )(acc, dnm, b2r)


def _tile_rows(n):
    even = ((n + 15) // 16 + 7) // 8 * 8
    return even, n - 15 * even


def _zero_vec(ref, c):
    def body(i, _):
        ref[pl.ds(i * LANES, LANES)] = jnp.zeros((LANES,), jnp.float32)
        return 0
    lax.fori_loop(0, c // LANES, body, 0)


def _zero_ivec(ref, c):
    def body(i, _):
        ref[pl.ds(i * LANES, LANES)] = jnp.zeros((LANES,), jnp.int32)
        return 0
    lax.fori_loop(0, c // LANES, body, 0)


def _slab_copy(src_at, dst_at, even, tail, s, chunk, via_at=None):
    def emit(total, base):
        off = 0
        while off < total:
            sz = min(chunk, total - off)
            if via_at is None:
                pltpu.sync_copy(src_at(base + off, sz), dst_at(base + off, sz))
            else:
                pltpu.sync_copy(src_at(base + off, sz), via_at(sz))
                pltpu.sync_copy(via_at(sz), dst_at(base + off, sz))
            off += sz

    @pl.when(s < 15)
    def _():
        emit(even, s * even)

    @pl.when(s == 15)
    def _():
        emit(tail, 15 * even)


def _make_attn(K, N, E, with_dnm):
    C = CS
    even, tail = _tile_rows(N)
    nch = E // C
    assert E % C == 0 and C % LANES == 0

    def body(*refs):
        srcs, dsts = refs[0], refs[1]
        els = refs[2:2 + K]
        ers = refs[2 + K:2 + 2 * K]
        cvec = refs[2 + 2 * K]
        ee_out = refs[3 + 2 * K]
        i = 5 + 2 * K if with_dnm else 4 + 2 * K
        dnm_out = refs[4 + 2 * K] if with_dnm else None
        src_v, dst_v = refs[i], refs[i + 1]
        elgs = refs[i + 2:i + 2 + K]
        ergs = refs[i + 2 + K:i + 2 + 2 * K]
        eevs = refs[i + 2 + 2 * K:i + 2 + 3 * K]
        cv, sem = refs[i + 2 + 3 * K], refs[i + 3 + 3 * K]
        dnm_sh = refs[i + 4 + 3 * K:i + 4 + 4 * K] if with_dnm else ()

        c = lax.axis_index("c")
        s = lax.axis_index("s")
        wid = c * 16 + s

        if with_dnm:
            _zero_vec(eevs[0], C)
            for k in range(K):
                _slab_copy(lambda o, z: eevs[0].at[pl.ds(0, z)],
                           lambda o, z, k=k: dnm_sh[k].at[pl.ds(o, z)],
                           even, tail, s, chunk=C)
            plsc.subcore_barrier()

        cvvs = []
        for k in range(K):
            pltpu.sync_copy(cvec.at[pl.ds(k * LANES, LANES)], cv)
            cvvs.append(cv[...])

        def chunk(j, _):
            base = (wid + j * 32) * C
            d1 = pltpu.async_copy(srcs.at[pl.ds(base, C)], src_v, sem)
            d2 = pltpu.async_copy(dsts.at[pl.ds(base, C)], dst_v, sem)
            d1.wait()
            d2.wait()
            dg = [pltpu.async_copy(els[k].at[src_v], elgs[k], sem)
                  for k in range(K)]
            dg += [pltpu.async_copy(ers[k].at[dst_v], ergs[k], sem)
                   for k in range(K)]
            for d in dg:
                d.wait()

            for k in range(K):
                cvv = cvvs[k]

                @plsc.parallel_loop(0, C // LANES, unroll=2)
                def _vr(i2, k=k, cvv=cvv):
                    z = (elgs[k][pl.ds(i2 * LANES, LANES)]
                         + ergs[k][pl.ds(i2 * LANES, LANES)])
                    z = jnp.where(z >= 0, z, NEG_SLOPE * z)
                    eevs[k][pl.ds(i2 * LANES, LANES)] = jnp.exp(z - cvv)

                pltpu.sync_copy(eevs[k], ee_out.at[pl.ds(k * E + base, C)])
                if with_dnm:
                    pltpu.sync_copy(eevs[k], dnm_sh[k].at[dst_v], add=True)
            return 0
        lax.fori_loop(0, (nch - wid + 31) // 32, chunk, 0)

        if with_dnm:
            plsc.subcore_barrier()
            for k in range(K):
                _slab_copy(lambda o, z, k=k: dnm_sh[k].at[pl.ds(o, z)],
                           lambda o, z, k=k: dnm_out.at[pl.ds((c * K + k) * N + o, z)],
                           even, tail, s, chunk=C,
                           via_at=lambda z: eevs[0].at[pl.ds(0, z)])

    out_type = [jax.ShapeDtypeStruct((K * E,), jnp.float32)]
    if with_dnm:
        out_type.append(jax.ShapeDtypeStruct((2 * K * N,), jnp.float32))
    scratch = ([pltpu.VMEM((C,), jnp.int32),
                pltpu.VMEM((C,), jnp.int32)]
               + [pltpu.VMEM((C,), jnp.float32)] * (3 * K)
               + [pltpu.VMEM((LANES,), jnp.float32),
                  pltpu.SemaphoreType.DMA])
    if with_dnm:
        scratch += [pltpu.VMEM_SHARED((N,), jnp.float32)] * K
    return pl.kernel(body, out_type=out_type,
                     mesh=plsc.VectorSubcoreMesh(**_MESH),
                     scratch_types=scratch, compiler_params=_CP)


def _make_agg(KE, N, E, segsets):
    C = CS
    even, tail = _tile_rows(NR)
    nch = E // C
    assert E % C == 0

    def body(*refs):
        srcs, dsts, tbl = refs[0], refs[1], refs[2]
        eerows = refs[3]
        acc_out = refs[4]
        src_vs = refs[5:7]
        dst_vs = refs[7:9]
        eevss = (refs[9:9 + KE], refs[9 + KE:9 + 2 * KE])
        r0 = 9 + 2 * KE
        st_src = refs[r0]
        st_dst = refs[r0 + 1]
        st_es = refs[r0 + 2:r0 + 2 + KE]
        fire_src = refs[r0 + 2 + KE]
        fire_dst = refs[r0 + 3 + KE]
        rows_v = refs[r0 + 4 + KE]
        acc_sh = refs[r0 + 5 + KE]
        sem = refs[r0 + 6 + KE]

        c = lax.axis_index("c")
        s = lax.axis_index("s")
        iota = lax.iota(jnp.int32, LANES)
        ntrips = (nch - s + 15) // 16

        def zrows(i, _):
            for half in range(8):
                rows_v[i, pl.ds(half * LANES, LANES)] = jnp.zeros((LANES,), jnp.float32)
            return 0

        _zero_ivec(st_src, CF + LANES)
        _zero_ivec(st_dst, CF + LANES)
        for k in range(KE):
            _zero_vec(st_es[k], CF + LANES)

        def issue(j, b):
            base = (s + j * 16) * C
            return ([pltpu.async_copy(srcs.at[pl.ds(base, C)], src_vs[b], sem),
                     pltpu.async_copy(dsts.at[pl.ds(base, C)], dst_vs[b], sem)]
                    + [pltpu.async_copy(eerows.at[pl.ds(k * E + base, C)],
                                        eevss[b][k], sem) for k in range(KE)])

        for phase in range(2):
            rid = 2 * phase + c
            lo = rid * NR

            lax.fori_loop(0, CF, zrows, 0)
            _slab_copy(lambda o, z: rows_v.at[pl.ds(0, z)],
                       lambda o, z: acc_sh.at[pl.ds(o, z)],
                       even, tail, s, chunk=CF)
            plsc.subcore_barrier()

            def fire(lo=lo):
                def mv(i, _):
                    fire_src[pl.ds(i * LANES, LANES)] = st_src[pl.ds(i * LANES, LANES)]
                    fire_dst[pl.ds(i * LANES, LANES)] = st_dst[pl.ds(i * LANES, LANES)]
                    return 0
                lax.fori_loop(0, CF // LANES, mv, 0)
                pltpu.sync_copy(tbl.at[fire_src], rows_v)

                @plsc.parallel_loop(0, CF // LANES, unroll=2)
                def _scale(i):
                    ees = [st_es[k][pl.ds(i * LANES, LANES)] for k in range(KE)]
                    for t in range(LANES):
                        tv = jnp.full((LANES,), t, jnp.int32)
                        sp = [e.at[tv].get(mode="promise_in_bounds") for e in ees]
                        for seg, kv in segsets(sp):
                            rr = rows_v[i * LANES + t, pl.ds(seg * LANES, LANES)]
                            rows_v[i * LANES + t, pl.ds(seg * LANES, LANES)] = rr * kv
                pltpu.sync_copy(rows_v, acc_sh.at[fire_dst], add=True)

            def process(b, pos, lo=lo, fire=fire):
                src_v, dst_v, eevs = src_vs[b], dst_vs[b], eevss[b]

                def vr(i, pos):
                    sv = src_v[pl.ds(i * LANES, LANES)]
                    dv = dst_v[pl.ds(i * LANES, LANES)]
                    dl = dv - lo
                    m = (dl >= 0) & (dl < NR)
                    plsc.store_compressed(st_src.at[pl.ds(pos, LANES)], sv, mask=m)
                    plsc.store_compressed(st_dst.at[pl.ds(pos, LANES)], dl, mask=m)
                    for k in range(KE):
                        ev = eevs[k][pl.ds(i * LANES, LANES)]
                        plsc.store_compressed(st_es[k].at[pl.ds(pos, LANES)], ev, mask=m)
                    pos2 = pos + jnp.sum(m.astype(jnp.int32))

                    @pl.when(pos2 >= CF)
                    def _():
                        fire()
                        v = st_src[pl.ds(CF, LANES)]
                        st_src[pl.ds(0, LANES)] = v
                        v = st_dst[pl.ds(CF, LANES)]
                        st_dst[pl.ds(0, LANES)] = v
                        for k in range(KE):
                            v = st_es[k][pl.ds(CF, LANES)]
                            st_es[k][pl.ds(0, LANES)] = v
                    return jnp.where(pos2 >= CF, pos2 - CF, pos2)
                return lax.fori_loop(0, C // LANES, vr, pos)

            def pair(p, pos, process=process):
                j0 = 2 * p
                d0 = issue(j0, 0)
                has2 = j0 + 1 < ntrips

                @pl.when(has2)
                def _():
                    issue(j0 + 1, 1)
                for d in d0:
                    d.wait()
                pos = process(0, pos)

                def second(pos):
                    pltpu.make_async_copy(srcs.at[pl.ds(0, C)], src_vs[1], sem).wait()
                    pltpu.make_async_copy(dsts.at[pl.ds(0, C)], dst_vs[1], sem).wait()
                    for k in range(KE):
                        pltpu.make_async_copy(eerows.at[pl.ds(0, C)],
                                              eevss[1][k], sem).wait()
                    return process(1, pos)
                return lax.cond(has2, second, lambda pos: pos, pos)

            pos = lax.fori_loop(0, (ntrips + 1) // 2, pair, 0)

            def ztail(i, p):
                gi = i * LANES + iota
                keep = gi < p
                for k in range(KE):
                    v = st_es[k][pl.ds(i * LANES, LANES)]
                    st_es[k][pl.ds(i * LANES, LANES)] = jnp.where(keep, v, 0.0)
                return p
            lax.fori_loop(0, CF // LANES, ztail, pos)
            fire()

            plsc.subcore_barrier()
            _slab_copy(lambda o, z: acc_sh.at[pl.ds(o, z)],
                       lambda o, z: acc_out.at[rid, pl.ds(o, z)],
                       even, tail, s, chunk=CF,
                       via_at=lambda z: rows_v.at[pl.ds(0, z)])
            plsc.subcore_barrier()

    scratch = ([pltpu.VMEM((C,), jnp.int32)] * 2
               + [pltpu.VMEM((C,), jnp.int32)] * 2
               + [pltpu.VMEM((C,), jnp.float32)] * (2 * KE)
               + [pltpu.VMEM((CF + LANES,), jnp.int32),
                  pltpu.VMEM((CF + LANES,), jnp.int32)]
               + [pltpu.VMEM((CF + LANES,), jnp.float32)] * KE
               + [pltpu.VMEM((CF,), jnp.int32),
                  pltpu.VMEM((CF,), jnp.int32),
                  pltpu.VMEM((CF, 128), jnp.float32),
                  pltpu.VMEM_SHARED((NR, 128), jnp.float32),
                  pltpu.SemaphoreType.DMA])
    return pl.kernel(body,
                     out_type=jax.ShapeDtypeStruct((4, NR, 128), jnp.float32),
                     mesh=plsc.VectorSubcoreMesh(**_MESH),
                     scratch_types=scratch, compiler_params=_CP)


def _headmat(al, heads, dim):
    m = jnp.zeros((heads * dim, heads), jnp.float32)
    return m.at[jnp.arange(heads * dim), jnp.repeat(jnp.arange(heads), dim)
                ].set(al.reshape(-1))


def _bound(el, er):
    z = jnp.max(el, axis=0) + jnp.max(er, axis=0)
    return jnp.where(z >= 0, z, NEG_SLOPE * z)


def _l1_segsets(sp):
    lane = lax.iota(jnp.int32, LANES)
    mix = jnp.where(lane == 0, sp[0], jnp.where(lane == 1, sp[1], 0.0))
    return [(0, sp[0]), (1, sp[0]), (2, sp[1]), (3, sp[1]), (4, mix)]


def _l2_segsets(sp):
    return [(seg, sp[0]) for seg in range(8)]


def kernel(x, edge_index, W1, al1, ar1, b1, W2, al2, ar2, b2):
    n = x.shape[0]
    e = edge_index.shape[1]
    ei = edge_index.astype(jnp.int32)
    srcs, dsts = ei[0], ei[1]

    t1, el, er = _tc1(x, W1, _headmat(al1, 2, 32), _headmat(ar1, 2, 32))
    cvec1 = jnp.repeat(_bound(el, er), LANES)

    (ee1,) = _make_attn(2, n, e, with_dnm=False)(
        srcs, dsts, el[:, 0], el[:, 1], er[:, 0], er[:, 1], cvec1)

    ph = _make_agg(2, n, e, _l1_segsets)(srcs, dsts, t1, ee1)
    acc1 = ph.reshape(4 * NR, 128)[:n]

    t2, el2, er2 = _tc2(acc1, b1.reshape(1, -1), W2,
                        _headmat(al2, 1, 128), _headmat(ar2, 1, 128))
    cvec2 = jnp.repeat(_bound(el2, er2), LANES)

    ee2, dnm2 = _make_attn(1, n, e, with_dnm=True)(
        srcs, dsts, el2[:, 0], er2[:, 0], cvec2)

    qh = _make_agg(1, n, e, _l2_segsets)(srcs, dsts, t2, ee2)
    acc2 = qh.reshape(4 * NR, 128)[:n]

    dnm2t = dnm2.reshape(2, n).transpose(1, 0)
    return _tc3(acc2, dnm2t, b2.reshape(1, -1))


# speedup vs baseline: 1.0315x; 1.0006x over previous
import jax
import jax.numpy as jnp
from jax import lax
from jax.experimental import pallas as pl
from jax.experimental.pallas import tpu as pltpu
from jax.experimental.pallas import tpu_sc as plsc

NEG_SLOPE = 0.2
LANES = 16
NR = 12512
CF = 128
CS = 1600

_MESH = dict(core_axis_name="c", subcore_axis_name="s")
_CP = pltpu.CompilerParams(needs_layout_passes=False)


def _elu(v):
    return jnp.where(v > 0, v, jnp.exp(jnp.minimum(v, 0.0)) - 1.0)


def _tc1_body(x_ref, w_ref, al_ref, ar_ref, t_ref, el_ref, er_ref):
    feat = jnp.dot(x_ref[...], w_ref[...], preferred_element_type=jnp.float32)
    bn = feat.shape[0]
    t_ref[...] = jnp.concatenate(
        [feat, jnp.ones((bn, 2), jnp.float32), jnp.zeros((bn, 62), jnp.float32)],
        axis=1)
    el_ref[...] = jnp.dot(feat, al_ref[...], preferred_element_type=jnp.float32)
    er_ref[...] = jnp.dot(feat, ar_ref[...], preferred_element_type=jnp.float32)


def _tc1(x, W1, almat, armat, bn=1000):
    n, din = x.shape
    h = almat.shape[1]
    full = lambda a: pl.BlockSpec(a.shape, lambda i: (0,) * a.ndim)
    return pl.pallas_call(
        _tc1_body,
        grid=(n // bn,),
        in_specs=[pl.BlockSpec((bn, din), lambda i: (i, 0)),
                  full(W1), full(almat), full(armat)],
        out_specs=[pl.BlockSpec((bn, 128), lambda i: (i, 0)),
                   pl.BlockSpec((bn, h), lambda i: (i, 0)),
                   pl.BlockSpec((bn, h), lambda i: (i, 0))],
        out_shape=[jax.ShapeDtypeStruct((n, 128), jnp.float32),
                   jax.ShapeDtypeStruct((n, h), jnp.float32),
                   jax.ShapeDtypeStruct((n, h), jnp.float32)],
    )(x, W1, almat, armat)


def _tc2_body(acc_ref, b1_ref, w2_ref, al_ref, ar_ref, t_ref, el_ref, er_ref):
    a0 = acc_ref[:, 0:32]
    a1 = acc_ref[:, 32:64]
    d0 = acc_ref[:, 64:65]
    d1 = acc_ref[:, 65:66]
    h0 = _elu(jnp.where(d0 > 0, a0 / d0, 0.0) + b1_ref[:, :32])
    h1 = _elu(jnp.where(d1 > 0, a1 / d1, 0.0) + b1_ref[:, 32:])
    h = jnp.concatenate([h0, h1], axis=1)
    feat = jnp.dot(h, w2_ref[...], preferred_element_type=jnp.float32)
    t_ref[...] = feat
    el_ref[...] = jnp.dot(feat, al_ref[...], preferred_element_type=jnp.float32)
    er_ref[...] = jnp.dot(feat, ar_ref[...], preferred_element_type=jnp.float32)


def _tc2(acc, b1r, W2, almat, armat, bn=1000):
    n = acc.shape[0]
    full = lambda a: pl.BlockSpec(a.shape, lambda i: (0,) * a.ndim)
    return pl.pallas_call(
        _tc2_body,
        grid=(n // bn,),
        in_specs=[pl.BlockSpec((bn, 128), lambda i: (i, 0)),
                  full(b1r), full(W2), full(almat), full(armat)],
        out_specs=[pl.BlockSpec((bn, 128), lambda i: (i, 0)),
                   pl.BlockSpec((bn, 1), lambda i: (i, 0)),
                   pl.BlockSpec((bn, 1), lambda i: (i, 0))],
        out_shape=[jax.ShapeDtypeStruct((n, 128), jnp.float32),
                   jax.ShapeDtypeStruct((n, 1), jnp.float32),
                   jax.ShapeDtypeStruct((n, 1), jnp.float32)],
    )(acc, b1r, W2, almat, armat)


def _tc3_body(acc_ref, dnm_ref, b2_ref, out_ref):
    d = (dnm_ref[:, 0] + dnm_ref[:, 1])[:, None]
    out_ref[...] = _elu(jnp.where(d > 0, acc_ref[...] / d, 0.0) + b2_ref[...])


def _tc3(acc, dnm, b2r, bn=1000):
    n = acc.shape[0]
    full = lambda a: pl.BlockSpec(a.shape, lambda i: (0,) * a.ndim)
    return pl.pallas_call(
        _tc3_body,
        grid=(n // bn,),
        in_specs=[pl.BlockSpec((bn, 128), lambda i: (i, 0)),
                  pl.BlockSpec((bn, 2), lambda i: (i, 0)), full(b2r)],
        out_specs=pl.BlockSpec((bn, 128), lambda i: (i, 0)),
        out_shape=jax.ShapeDtypeStruct((n, 128), jnp.float32),
    )(acc, dnm, b2r)


def _tile_rows(n):
    even = ((n + 15) // 16 + 7) // 8 * 8
    return even, n - 15 * even


def _zero_vec(ref, c):
    def body(i, _):
        ref[pl.ds(i * LANES, LANES)] = jnp.zeros((LANES,), jnp.float32)
        return 0
    lax.fori_loop(0, c // LANES, body, 0)


def _zero_ivec(ref, c):
    def body(i, _):
        ref[pl.ds(i * LANES, LANES)] = jnp.zeros((LANES,), jnp.int32)
        return 0
    lax.fori_loop(0, c // LANES, body, 0)


def _slab_copy(src_at, dst_at, even, tail, s, chunk, via_at=None):
    def emit(total, base):
        off = 0
        while off < total:
            sz = min(chunk, total - off)
            if via_at is None:
                pltpu.sync_copy(src_at(base + off, sz), dst_at(base + off, sz))
            else:
                pltpu.sync_copy(src_at(base + off, sz), via_at(sz))
                pltpu.sync_copy(via_at(sz), dst_at(base + off, sz))
            off += sz

    @pl.when(s < 15)
    def _():
        emit(even, s * even)

    @pl.when(s == 15)
    def _():
        emit(tail, 15 * even)


def _make_attn(K, N, E, with_dnm):
    C = CS
    even, tail = _tile_rows(N)
    nch = E // C
    assert E % C == 0 and C % LANES == 0

    def body(*refs):
        srcs, dsts = refs[0], refs[1]
        els = refs[2:2 + K]
        ers = refs[2 + K:2 + 2 * K]
        cvec = refs[2 + 2 * K]
        ee_out = refs[3 + 2 * K]
        i = 5 + 2 * K if with_dnm else 4 + 2 * K
        dnm_out = refs[4 + 2 * K] if with_dnm else None
        src_v, dst_v = refs[i], refs[i + 1]
        elgs = refs[i + 2:i + 2 + K]
        ergs = refs[i + 2 + K:i + 2 + 2 * K]
        eevs = refs[i + 2 + 2 * K:i + 2 + 3 * K]
        cv, sem = refs[i + 2 + 3 * K], refs[i + 3 + 3 * K]
        dnm_sh = refs[i + 4 + 3 * K:i + 4 + 4 * K] if with_dnm else ()

        c = lax.axis_index("c")
        s = lax.axis_index("s")
        wid = c * 16 + s

        if with_dnm:
            _zero_vec(eevs[0], C)
            for k in range(K):
                _slab_copy(lambda o, z: eevs[0].at[pl.ds(0, z)],
                           lambda o, z, k=k: dnm_sh[k].at[pl.ds(o, z)],
                           even, tail, s, chunk=C)
            plsc.subcore_barrier()

        cvvs = []
        for k in range(K):
            pltpu.sync_copy(cvec.at[pl.ds(k * LANES, LANES)], cv)
            cvvs.append(cv[...])

        def chunk(j, _):
            base = (wid + j * 32) * C
            d1 = pltpu.async_copy(srcs.at[pl.ds(base, C)], src_v, sem)
            d2 = pltpu.async_copy(dsts.at[pl.ds(base, C)], dst_v, sem)
            d1.wait()
            d2.wait()
            dg = [pltpu.async_copy(els[k].at[src_v], elgs[k], sem)
                  for k in range(K)]
            dg += [pltpu.async_copy(ers[k].at[dst_v], ergs[k], sem)
                   for k in range(K)]
            for d in dg:
                d.wait()

            for k in range(K):
                cvv = cvvs[k]

                @plsc.parallel_loop(0, C // LANES, unroll=2)
                def _vr(i2, k=k, cvv=cvv):
                    z = (elgs[k][pl.ds(i2 * LANES, LANES)]
                         + ergs[k][pl.ds(i2 * LANES, LANES)])
                    z = jnp.where(z >= 0, z, NEG_SLOPE * z)
                    eevs[k][pl.ds(i2 * LANES, LANES)] = jnp.exp(z - cvv)

                pltpu.sync_copy(eevs[k], ee_out.at[pl.ds(k * E + base, C)])
                if with_dnm:
                    pltpu.sync_copy(eevs[k], dnm_sh[k].at[dst_v], add=True)
            return 0
        lax.fori_loop(0, (nch - wid + 31) // 32, chunk, 0)

        if with_dnm:
            plsc.subcore_barrier()
            for k in range(K):
                _slab_copy(lambda o, z, k=k: dnm_sh[k].at[pl.ds(o, z)],
                           lambda o, z, k=k: dnm_out.at[pl.ds((c * K + k) * N + o, z)],
                           even, tail, s, chunk=C,
                           via_at=lambda z: eevs[0].at[pl.ds(0, z)])

    out_type = [jax.ShapeDtypeStruct((K * E,), jnp.float32)]
    if with_dnm:
        out_type.append(jax.ShapeDtypeStruct((2 * K * N,), jnp.float32))
    scratch = ([pltpu.VMEM((C,), jnp.int32),
                pltpu.VMEM((C,), jnp.int32)]
               + [pltpu.VMEM((C,), jnp.float32)] * (3 * K)
               + [pltpu.VMEM((LANES,), jnp.float32),
                  pltpu.SemaphoreType.DMA])
    if with_dnm:
        scratch += [pltpu.VMEM_SHARED((N,), jnp.float32)] * K
    return pl.kernel(body, out_type=out_type,
                     mesh=plsc.VectorSubcoreMesh(**_MESH),
                     scratch_types=scratch, compiler_params=_CP)


def _make_agg(KE, N, E, segsets):
    C = CS
    even, tail = _tile_rows(NR)
    nch = E // C
    assert E % C == 0

    def body(*refs):
        srcs, dsts, tbl = refs[0], refs[1], refs[2]
        eerows = refs[3]
        acc_out = refs[4]
        src_vs = refs[5:7]
        dst_vs = refs[7:9]
        eevss = (refs[9:9 + KE], refs[9 + KE:9 + 2 * KE])
        r0 = 9 + 2 * KE
        st_src = refs[r0]
        st_dst = refs[r0 + 1]
        st_es = refs[r0 + 2:r0 + 2 + KE]
        fire_src = refs[r0 + 2 + KE]
        fire_dst = refs[r0 + 3 + KE]
        rows_v = refs[r0 + 4 + KE]
        acc_sh = refs[r0 + 5 + KE]
        sem = refs[r0 + 6 + KE]

        c = lax.axis_index("c")
        s = lax.axis_index("s")
        iota = lax.iota(jnp.int32, LANES)
        ntrips = (nch - s + 15) // 16

        def zrows(i, _):
            for half in range(8):
                rows_v[i, pl.ds(half * LANES, LANES)] = jnp.zeros((LANES,), jnp.float32)
            return 0

        _zero_ivec(st_src, CF + LANES)
        _zero_ivec(st_dst, CF + LANES)
        for k in range(KE):
            _zero_vec(st_es[k], CF + LANES)

        def issue(j, b):
            base = (s + j * 16) * C
            return ([pltpu.async_copy(srcs.at[pl.ds(base, C)], src_vs[b], sem),
                     pltpu.async_copy(dsts.at[pl.ds(base, C)], dst_vs[b], sem)]
                    + [pltpu.async_copy(eerows.at[pl.ds(k * E + base, C)],
                                        eevss[b][k], sem) for k in range(KE)])

        for phase in range(2):
            rid = 2 * phase + c
            lo = rid * NR

            lax.fori_loop(0, CF, zrows, 0)
            _slab_copy(lambda o, z: rows_v.at[pl.ds(0, z)],
                       lambda o, z: acc_sh.at[pl.ds(o, z)],
                       even, tail, s, chunk=CF)
            plsc.subcore_barrier()

            def fire(lo=lo):
                def mv(i, _):
                    fire_src[pl.ds(i * LANES, LANES)] = st_src[pl.ds(i * LANES, LANES)]
                    fire_dst[pl.ds(i * LANES, LANES)] = st_dst[pl.ds(i * LANES, LANES)]
                    return 0
                lax.fori_loop(0, CF // LANES, mv, 0)
                pltpu.sync_copy(tbl.at[fire_src], rows_v)

                @plsc.parallel_loop(0, CF // LANES, unroll=2)
                def _scale(i):
                    ees = [st_es[k][pl.ds(i * LANES, LANES)] for k in range(KE)]
                    for t in range(LANES):
                        tv = jnp.full((LANES,), t, jnp.int32)
                        sp = [e.at[tv].get(mode="promise_in_bounds") for e in ees]
                        for seg, kv in segsets(sp):
                            rr = rows_v[i * LANES + t, pl.ds(seg * LANES, LANES)]
                            rows_v[i * LANES + t, pl.ds(seg * LANES, LANES)] = rr * kv
                pltpu.sync_copy(rows_v, acc_sh.at[fire_dst], add=True)

            def process(b, pos, lo=lo, fire=fire):
                src_v, dst_v, eevs = src_vs[b], dst_vs[b], eevss[b]

                def vr(i, pos):
                    sv = src_v[pl.ds(i * LANES, LANES)]
                    dv = dst_v[pl.ds(i * LANES, LANES)]
                    dl = dv - lo
                    m = (dl >= 0) & (dl < NR)
                    plsc.store_compressed(st_src.at[pl.ds(pos, LANES)], sv, mask=m)
                    plsc.store_compressed(st_dst.at[pl.ds(pos, LANES)], dl, mask=m)
                    for k in range(KE):
                        ev = eevs[k][pl.ds(i * LANES, LANES)]
                        plsc.store_compressed(st_es[k].at[pl.ds(pos, LANES)], ev, mask=m)
                    pos2 = pos + jnp.sum(m.astype(jnp.int32))

                    @pl.when(pos2 >= CF)
                    def _():
                        fire()
                        v = st_src[pl.ds(CF, LANES)]
                        st_src[pl.ds(0, LANES)] = v
                        v = st_dst[pl.ds(CF, LANES)]
                        st_dst[pl.ds(0, LANES)] = v
                        for k in range(KE):
                            v = st_es[k][pl.ds(CF, LANES)]
                            st_es[k][pl.ds(0, LANES)] = v
                    return jnp.where(pos2 >= CF, pos2 - CF, pos2)
                return lax.fori_loop(0, C // LANES, vr, pos)

            def pair(p, pos, process=process):
                j0 = 2 * p
                d0 = issue(j0, 0)
                has2 = j0 + 1 < ntrips

                @pl.when(has2)
                def _():
                    issue(j0 + 1, 1)
                for d in d0:
                    d.wait()
                pos = process(0, pos)

                def second(pos):
                    pltpu.make_async_copy(srcs.at[pl.ds(0, C)], src_vs[1], sem).wait()
                    pltpu.make_async_copy(dsts.at[pl.ds(0, C)], dst_vs[1], sem).wait()
                    for k in range(KE):
                        pltpu.make_async_copy(eerows.at[pl.ds(0, C)],
                                              eevss[1][k], sem).wait()
                    return process(1, pos)
                return lax.cond(has2, second, lambda pos: pos, pos)

            pos = lax.fori_loop(0, (ntrips + 1) // 2, pair, 0)

            def ztail(i, p):
                gi = i * LANES + iota
                keep = gi < p
                for k in range(KE):
                    v = st_es[k][pl.ds(i * LANES, LANES)]
                    st_es[k][pl.ds(i * LANES, LANES)] = jnp.where(keep, v, 0.0)
                return p
            lax.fori_loop(0, CF // LANES, ztail, pos)
            fire()

            plsc.subcore_barrier()
            _slab_copy(lambda o, z: acc_sh.at[pl.ds(o, z)],
                       lambda o, z: acc_out.at[rid, pl.ds(o, z)],
                       even, tail, s, chunk=CF,
                       via_at=lambda z: rows_v.at[pl.ds(0, z)])
            plsc.subcore_barrier()

    scratch = ([pltpu.VMEM((C,), jnp.int32)] * 2
               + [pltpu.VMEM((C,), jnp.int32)] * 2
               + [pltpu.VMEM((C,), jnp.float32)] * (2 * KE)
               + [pltpu.VMEM((CF + LANES,), jnp.int32),
                  pltpu.VMEM((CF + LANES,), jnp.int32)]
               + [pltpu.VMEM((CF + LANES,), jnp.float32)] * KE
               + [pltpu.VMEM((CF,), jnp.int32),
                  pltpu.VMEM((CF,), jnp.int32),
                  pltpu.VMEM((CF, 128), jnp.float32),
                  pltpu.VMEM_SHARED((NR, 128), jnp.float32),
                  pltpu.SemaphoreType.DMA])
    return pl.kernel(body,
                     out_type=jax.ShapeDtypeStruct((4, NR, 128), jnp.float32),
                     mesh=plsc.VectorSubcoreMesh(**_MESH),
                     scratch_types=scratch, compiler_params=_CP)


def _headmat(al, heads, dim):
    m = jnp.zeros((heads * dim, heads), jnp.float32)
    return m.at[jnp.arange(heads * dim), jnp.repeat(jnp.arange(heads), dim)
                ].set(al.reshape(-1))


def _bound(el, er):
    z = jnp.max(el, axis=0) + jnp.max(er, axis=0)
    return jnp.where(z >= 0, z, NEG_SLOPE * z)


def _l1_segsets(sp):
    lane = lax.iota(jnp.int32, LANES)
    mix = jnp.where(lane == 0, sp[0], jnp.where(lane == 1, sp[1], 0.0))
    return [(0, sp[0]), (1, sp[0]), (2, sp[1]), (3, sp[1]), (4, mix)]


def _l2_segsets(sp):
    return [(seg, sp[0]) for seg in range(8)]


def kernel(x, edge_index, W1, al1, ar1, b1, W2, al2, ar2, b2):
    n = x.shape[0]
    e = edge_index.shape[1]
    ei = edge_index.astype(jnp.int32)
    srcs, dsts = ei[0], ei[1]

    t1, el, er = _tc1(x, W1, _headmat(al1, 2, 32), _headmat(ar1, 2, 32))
    cvec1 = jnp.repeat(_bound(el, er), LANES)

    (ee1,) = _make_attn(2, n, e, with_dnm=False)(
        srcs, dsts, el[:, 0], el[:, 1], er[:, 0], er[:, 1], cvec1)

    ph = _make_agg(2, n, e, _l1_segsets)(srcs, dsts, t1, ee1)
    acc1 = ph.reshape(4 * NR, 128)[:n]

    t2, el2, er2 = _tc2(acc1, b1.reshape(1, -1), W2,
                        _headmat(al2, 1, 128), _headmat(ar2, 1, 128))
    cvec2 = jnp.repeat(_bound(el2, er2), LANES)

    ee2, dnm2 = _make_attn(1, n, e, with_dnm=True)(
        srcs, dsts, el2[:, 0], er2[:, 0], cvec2)

    qh = _make_agg(1, n, e, _l2_segsets)(srcs, dsts, t2, ee2)
    acc2 = qh.reshape(4 * NR, 128)[:n]

    dnm2t = dnm2.reshape(2, n).transpose(1, 0)
    return _tc3(acc2, dnm2t, b2.reshape(1, -1))
